# Initial kernel scaffold; baseline (speedup 1.0000x reference)
import jax
import jax.numpy as jnp
from jax import lax
from jax.experimental import pallas as pl
from jax.experimental.pallas import tpu as pltpu
from jax.experimental.pallas import tpu_sc as plsc

_SH_C0 = 0.28209479177387814
_SH_C1 = 0.4886025119029199
_SH_C2 = (1.0925484305920792, -1.0925484305920792, 0.31539156525252005,
          -1.0925484305920792, 0.5462742152960396)

_SCALE = 1.5
_G = 128
_P = 524288
_C = 28

_NC = 2
_NS = 16
_NW = _NC * _NS
_BC = 128
_PPW = _P // _NW
_NCHUNK = _PPW // _BC

_LANES = 16
_NGRP = _BC // _LANES


def _axis_setup(coord):
    g = coord / _SCALE
    ix = ((g + 1.0) * float(_G) - 1.0) * 0.5
    ixc = jnp.clip(ix, -1.0, float(_G))
    t = ixc.astype(jnp.int32)
    tf = t.astype(jnp.float32)
    fl = tf - jnp.where(tf > ixc, 1.0, 0.0)
    fli = fl.astype(jnp.int32)
    frac = ixc - fl
    v0 = (ix >= 0.0) & (ix < float(_G))
    v1 = (ix >= -1.0) & (ix < float(_G) - 1.0)
    c0 = jnp.clip(fli, 0, _G - 1)
    c1 = jnp.clip(fli + 1, 0, _G - 1)
    w0 = jnp.where(v0, 1.0 - frac, 0.0)
    w1 = jnp.where(v1, frac, 0.0)
    return c0, c1, w0, w1


def _sc_body(x_hbm, d_hbm, tab_hbm, color_hbm, sigma_hbm,
             x_v, d_v, idx_v, w_v, g_v, color_v, sigma_v, sem):
    wid = lax.axis_index("s") * _NC + lax.axis_index("c")
    lanes = lax.iota(jnp.int32, _LANES)
    zeros16 = jnp.zeros((_LANES,), jnp.int32)

    def chunk_body(ci, carry):
        pbase = wid * _PPW + ci * _BC

        pltpu.sync_copy(x_hbm.at[pl.ds(pbase, _BC)], x_v)
        pltpu.sync_copy(d_hbm.at[pl.ds(pbase, _BC)], d_v)

        def prep_group(g, c2):
            base = g * _LANES
            rows = base + lanes
            px = plsc.load_gather(x_v, [rows, zeros16])
            py = plsc.load_gather(x_v, [rows, zeros16 + 1])
            pz = plsc.load_gather(x_v, [rows, zeros16 + 2])

            keep = ((px < _SCALE) & (px > -_SCALE) &
                    (py < _SCALE) & (py > -_SCALE) &
                    (pz < _SCALE) & (pz > -_SCALE))
            keep = keep | ((pbase + rows) == 0)
            keepf = jnp.where(keep, 1.0, 0.0)

            cx0, cx1, wx0, wx1 = _axis_setup(px)
            cy0, cy1, wy0, wy1 = _axis_setup(py)
            cz0, cz1, wz0, wz1 = _axis_setup(pz)
            wz0 = wz0 * keepf
            wz1 = wz1 * keepf

            z0 = cz0 * (_G * _G)
            z1 = cz1 * (_G * _G)
            y0 = cy0 * _G
            y1 = cy1 * _G
            wzy = (wz0 * wy0, wz0 * wy1, wz1 * wy0, wz1 * wy1)
            zy = (z0 + y0, z0 + y1, z1 + y0, z1 + y1)
            k = 0
            for j in range(4):
                for (cx, wx) in ((cx0, wx0), (cx1, wx1)):
                    idx_v[k, pl.ds(base, _LANES)] = zy[j] + cx
                    w_v[k, pl.ds(base, _LANES)] = wzy[j] * wx
                    k += 1
            return c2

        lax.fori_loop(0, _NGRP, prep_group, 0)

        descs = [
            pltpu.async_copy(tab_hbm.at[idx_v.at[k]],
                             g_v.at[pl.ds(k * _BC, _BC)], sem)
            for k in range(8)
        ]
        for dsc in descs:
            dsc.wait()

        def compute_group(g, c2):
            base = g * _LANES
            rows = base + lanes
            dx = plsc.load_gather(d_v, [rows, zeros16])
            dy = plsc.load_gather(d_v, [rows, zeros16 + 1])
            dz = plsc.load_gather(d_v, [rows, zeros16 + 2])
            xx, yy, zz = dx * dx, dy * dy, dz * dz
            basis = (
                jnp.full((_LANES,), _SH_C0, jnp.float32),
                (-_SH_C1) * dy,
                _SH_C1 * dz,
                (-_SH_C1) * dx,
                _SH_C2[0] * (dx * dy),
                _SH_C2[1] * (dy * dz),
                _SH_C2[2] * (2.0 * zz - xx - yy),
                _SH_C2[3] * (dx * dz),
                _SH_C2[4] * (xx - yy),
            )
            w = [w_v[k, pl.ds(base, _LANES)] for k in range(8)]
            rowk = [k * _BC + rows for k in range(8)]

            def interp(ch):
                col = zeros16 + ch
                acc = w[0] * plsc.load_gather(g_v, [rowk[0], col])
                for k in range(1, 8):
                    acc = acc + w[k] * plsc.load_gather(g_v, [rowk[k], col])
                return acc

            sigma = jnp.maximum(interp(0), 0.0)
            sigma_v[pl.ds(base, _LANES)] = sigma

            for c in range(3):
                acc = basis[0] * interp(1 + c * 9)
                for b in range(1, 9):
                    acc = acc + basis[b] * interp(1 + c * 9 + b)
                plsc.store_scatter(color_v, [rows, zeros16 + c], acc)
            return c2

        lax.fori_loop(0, _NGRP, compute_group, 0)

        pltpu.sync_copy(color_v, color_hbm.at[pl.ds(pbase, _BC)])
        pltpu.sync_copy(sigma_v, sigma_hbm.at[pl.ds(pbase, _BC)])
        return carry

    lax.fori_loop(0, _NCHUNK, chunk_body, 0)


@jax.jit
def kernel(x, d, voxel_grid):
    tab = voxel_grid.reshape(_G * _G * _G, _C)
    mesh = plsc.VectorSubcoreMesh(core_axis_name="c", subcore_axis_name="s",
                                  num_cores=_NC, num_subcores=_NS)
    fn = pl.kernel(
        _sc_body,
        out_type=(
            jax.ShapeDtypeStruct((_P, 3), jnp.float32),
            jax.ShapeDtypeStruct((_P,), jnp.float32),
        ),
        mesh=mesh,
        scratch_types=[
            pltpu.VMEM((_BC, 3), jnp.float32),
            pltpu.VMEM((_BC, 3), jnp.float32),
            pltpu.VMEM((8, _BC), jnp.int32),
            pltpu.VMEM((8, _BC), jnp.float32),
            pltpu.VMEM((8 * _BC, _C), jnp.float32),
            pltpu.VMEM((_BC, 3), jnp.float32),
            pltpu.VMEM((_BC,), jnp.float32),
            pltpu.SemaphoreType.DMA,
        ],
    )
    color, sigma = fn(x, d, tab)
    return (color, sigma)


# speedup vs baseline: 2.3082x; 2.3082x over previous
import jax
import jax.numpy as jnp
from jax import lax
from jax.experimental import pallas as pl
from jax.experimental.pallas import tpu as pltpu
from jax.experimental.pallas import tpu_sc as plsc

_SH_C0 = 0.28209479177387814
_SH_C1 = 0.4886025119029199
_SH_C2 = (1.0925484305920792, -1.0925484305920792, 0.31539156525252005,
          -1.0925484305920792, 0.5462742152960396)

_SCALE = 1.5
_G = 128
_P = 524288
_C = 28
_CT = 32

_NC = 2
_NS = 16
_NW = _NC * _NS
_BC = 128
_PPW = _P // _NW
_NCHUNK = _PPW // _BC

_LANES = 16
_NGRP = _BC // _LANES


def _axis_setup(coord):
    g = coord / _SCALE
    ix = ((g + 1.0) * float(_G) - 1.0) * 0.5
    ixc = jnp.clip(ix, -1.0, float(_G))
    t = ixc.astype(jnp.int32)
    tf = t.astype(jnp.float32)
    fl = tf - jnp.where(tf > ixc, 1.0, 0.0)
    fli = fl.astype(jnp.int32)
    frac = ixc - fl
    v0 = (ix >= 0.0) & (ix < float(_G))
    v1 = (ix >= -1.0) & (ix < float(_G) - 1.0)
    c0 = jnp.clip(fli, 0, _G - 1)
    c1 = jnp.clip(fli + 1, 0, _G - 1)
    w0 = jnp.where(v0, 1.0 - frac, 0.0)
    w1 = jnp.where(v1, frac, 0.0)
    return c0, c1, w0, w1


def _sc_body(x_hbm, d_hbm, tab_hbm, color_hbm, sigma_hbm,
             x_v, d_v, idx_v, w_v, g_v, color_v, sigma_v, sem):
    wid = lax.axis_index("s") * _NC + lax.axis_index("c")
    lanes = lax.iota(jnp.int32, _LANES)
    zeros16 = jnp.zeros((_LANES,), jnp.int32)

    def chunk_body(ci, carry):
        pbase = wid * _PPW + ci * _BC

        pltpu.sync_copy(x_hbm.at[pl.ds(pbase, _BC)], x_v)
        pltpu.sync_copy(d_hbm.at[pl.ds(pbase, _BC)], d_v)

        def prep_group(g, c2):
            base = g * _LANES
            rows = base + lanes
            px = plsc.load_gather(x_v, [rows, zeros16])
            py = plsc.load_gather(x_v, [rows, zeros16 + 1])
            pz = plsc.load_gather(x_v, [rows, zeros16 + 2])

            keep = ((px < _SCALE) & (px > -_SCALE) &
                    (py < _SCALE) & (py > -_SCALE) &
                    (pz < _SCALE) & (pz > -_SCALE))
            keep = keep | ((pbase + rows) == 0)
            keepf = jnp.where(keep, 1.0, 0.0)

            cx0, cx1, wx0, wx1 = _axis_setup(px)
            cy0, cy1, wy0, wy1 = _axis_setup(py)
            cz0, cz1, wz0, wz1 = _axis_setup(pz)
            wz0 = wz0 * keepf
            wz1 = wz1 * keepf

            z0 = cz0 * (_G * _G)
            z1 = cz1 * (_G * _G)
            y0 = cy0 * _G
            y1 = cy1 * _G
            wzy = (wz0 * wy0, wz0 * wy1, wz1 * wy0, wz1 * wy1)
            zy = (z0 + y0, z0 + y1, z1 + y0, z1 + y1)
            k = 0
            for j in range(4):
                for (cx, wx) in ((cx0, wx0), (cx1, wx1)):
                    idx_v[k, pl.ds(base, _LANES)] = zy[j] + cx
                    w_v[k, pl.ds(base, _LANES)] = wzy[j] * wx
                    k += 1
            return c2

        lax.fori_loop(0, _NGRP, prep_group, 0)

        descs = [
            pltpu.async_copy(tab_hbm.at[idx_v.at[k]],
                             g_v.at[pl.ds(k * _BC, _BC)], sem)
            for k in range(8)
        ]
        for dsc in descs:
            dsc.wait()

        def compute_group(g, c2):
            base = g * _LANES
            rows = base + lanes
            dx = plsc.load_gather(d_v, [rows, zeros16])
            dy = plsc.load_gather(d_v, [rows, zeros16 + 1])
            dz = plsc.load_gather(d_v, [rows, zeros16 + 2])
            xx, yy, zz = dx * dx, dy * dy, dz * dz
            basis = (
                jnp.full((_LANES,), _SH_C0, jnp.float32),
                (-_SH_C1) * dy,
                _SH_C1 * dz,
                (-_SH_C1) * dx,
                _SH_C2[0] * (dx * dy),
                _SH_C2[1] * (dy * dz),
                _SH_C2[2] * (2.0 * zz - xx - yy),
                _SH_C2[3] * (dx * dz),
                _SH_C2[4] * (xx - yy),
            )
            w = [w_v[k, pl.ds(base, _LANES)] for k in range(8)]
            rowk = [k * _BC + rows for k in range(8)]

            def interp(ch):
                col = zeros16 + ch
                acc = w[0] * plsc.load_gather(g_v, [rowk[0], col])
                for k in range(1, 8):
                    acc = acc + w[k] * plsc.load_gather(g_v, [rowk[k], col])
                return acc

            sigma = jnp.maximum(interp(0), 0.0)
            sigma_v[pl.ds(base, _LANES)] = sigma

            for c in range(3):
                acc = basis[0] * interp(1 + c * 9)
                for b in range(1, 9):
                    acc = acc + basis[b] * interp(1 + c * 9 + b)
                plsc.store_scatter(color_v, [rows, zeros16 + c], acc)
            return c2

        lax.fori_loop(0, _NGRP, compute_group, 0)

        pltpu.sync_copy(color_v, color_hbm.at[pl.ds(pbase, _BC)])
        pltpu.sync_copy(sigma_v, sigma_hbm.at[pl.ds(pbase, _BC)])
        return carry

    lax.fori_loop(0, _NCHUNK, chunk_body, 0)


@jax.jit
def kernel(x, d, voxel_grid):
    tab = jnp.pad(voxel_grid.reshape(_G * _G * _G, _C),
                  ((0, 0), (0, _CT - _C)))
    mesh = plsc.VectorSubcoreMesh(core_axis_name="c", subcore_axis_name="s",
                                  num_cores=_NC, num_subcores=_NS)
    fn = pl.kernel(
        _sc_body,
        out_type=(
            jax.ShapeDtypeStruct((_P, 3), jnp.float32),
            jax.ShapeDtypeStruct((_P,), jnp.float32),
        ),
        mesh=mesh,
        compiler_params=pltpu.CompilerParams(needs_layout_passes=False,
                                             use_tc_tiling_on_sc=False),
        scratch_types=[
            pltpu.VMEM((_BC, 3), jnp.float32),
            pltpu.VMEM((_BC, 3), jnp.float32),
            pltpu.VMEM((8, _BC), jnp.int32),
            pltpu.VMEM((8, _BC), jnp.float32),
            pltpu.VMEM((8 * _BC, _CT), jnp.float32),
            pltpu.VMEM((_BC, 3), jnp.float32),
            pltpu.VMEM((_BC,), jnp.float32),
            pltpu.SemaphoreType.DMA,
        ],
    )
    color, sigma = fn(x, d, tab)
    return (color, sigma)


# speedup vs baseline: 3.1628x; 1.3702x over previous
import jax
import jax.numpy as jnp
from jax import lax
from jax.experimental import pallas as pl
from jax.experimental.pallas import tpu as pltpu
from jax.experimental.pallas import tpu_sc as plsc

_SH_C0 = 0.28209479177387814
_SH_C1 = 0.4886025119029199
_SH_C2 = (1.0925484305920792, -1.0925484305920792, 0.31539156525252005,
          -1.0925484305920792, 0.5462742152960396)

_SCALE = 1.5
_G = 128
_P = 524288
_C = 28
_CT = 32

_NC = 2
_NS = 16
_NW = _NC * _NS
_BC = 128
_PPW = _P // _NW
_NCHUNK = _PPW // _BC

_LANES = 16
_NGRP = _BC // _LANES


def _axis_setup(coord):
    g = coord / _SCALE
    ix = ((g + 1.0) * float(_G) - 1.0) * 0.5
    ixc = jnp.clip(ix, -1.0, float(_G))
    t = ixc.astype(jnp.int32)
    tf = t.astype(jnp.float32)
    fl = tf - jnp.where(tf > ixc, 1.0, 0.0)
    fli = fl.astype(jnp.int32)
    frac = ixc - fl
    v0 = (ix >= 0.0) & (ix < float(_G))
    v1 = (ix >= -1.0) & (ix < float(_G) - 1.0)
    c0 = jnp.clip(fli, 0, _G - 1)
    c1 = jnp.clip(fli + 1, 0, _G - 1)
    w0 = jnp.where(v0, 1.0 - frac, 0.0)
    w1 = jnp.where(v1, frac, 0.0)
    return c0, c1, w0, w1


def _sc_body(x_hbm, d_hbm, tab_hbm, color_hbm, sigma_hbm,
             x_v, d_v, idx_v, w_v, g_v, color_v, sigma_v, sem):
    wid = lax.axis_index("s") * _NC + lax.axis_index("c")
    lanes = lax.iota(jnp.int32, _LANES)
    zeros16 = jnp.zeros((_LANES,), jnp.int32)

    def chunk_body(ci, carry):
        pbase = wid * _PPW + ci * _BC

        for c in range(3):
            pltpu.sync_copy(x_hbm.at[pl.ds(c * _P + pbase, _BC)], x_v.at[c])
            pltpu.sync_copy(d_hbm.at[pl.ds(c * _P + pbase, _BC)], d_v.at[c])

        def prep_group(g, c2):
            base = g * _LANES
            px = x_v[0, pl.ds(base, _LANES)]
            py = x_v[1, pl.ds(base, _LANES)]
            pz = x_v[2, pl.ds(base, _LANES)]
            rows = base + lanes

            keep = ((px < _SCALE) & (px > -_SCALE) &
                    (py < _SCALE) & (py > -_SCALE) &
                    (pz < _SCALE) & (pz > -_SCALE))
            keep = keep | ((pbase + rows) == 0)
            keepf = jnp.where(keep, 1.0, 0.0)

            cx0, cx1, wx0, wx1 = _axis_setup(px)
            cy0, cy1, wy0, wy1 = _axis_setup(py)
            cz0, cz1, wz0, wz1 = _axis_setup(pz)
            wz0 = wz0 * keepf
            wz1 = wz1 * keepf

            z0 = cz0 * (_G * _G)
            z1 = cz1 * (_G * _G)
            y0 = cy0 * _G
            y1 = cy1 * _G
            wzy = (wz0 * wy0, wz0 * wy1, wz1 * wy0, wz1 * wy1)
            zy = (z0 + y0, z0 + y1, z1 + y0, z1 + y1)
            k = 0
            for j in range(4):
                for (cx, wx) in ((cx0, wx0), (cx1, wx1)):
                    idx_v[k, pl.ds(base, _LANES)] = zy[j] + cx
                    w_v[k, pl.ds(base, _LANES)] = wzy[j] * wx
                    k += 1
            return c2

        lax.fori_loop(0, _NGRP, prep_group, 0)

        descs = [
            pltpu.async_copy(tab_hbm.at[idx_v.at[k]],
                             g_v.at[pl.ds(k * _BC, _BC)], sem)
            for k in range(8)
        ]
        for dsc in descs:
            dsc.wait()

        def compute_group(g, c2):
            base = g * _LANES
            rows = base + lanes
            dx = d_v[0, pl.ds(base, _LANES)]
            dy = d_v[1, pl.ds(base, _LANES)]
            dz = d_v[2, pl.ds(base, _LANES)]
            xx, yy, zz = dx * dx, dy * dy, dz * dz
            basis = (
                jnp.full((_LANES,), _SH_C0, jnp.float32),
                (-_SH_C1) * dy,
                _SH_C1 * dz,
                (-_SH_C1) * dx,
                _SH_C2[0] * (dx * dy),
                _SH_C2[1] * (dy * dz),
                _SH_C2[2] * (2.0 * zz - xx - yy),
                _SH_C2[3] * (dx * dz),
                _SH_C2[4] * (xx - yy),
            )
            w = [w_v[k, pl.ds(base, _LANES)] for k in range(8)]
            rowk = [k * _BC + rows for k in range(8)]

            def interp(ch):
                col = zeros16 + ch
                acc = w[0] * plsc.load_gather(g_v, [rowk[0], col])
                for k in range(1, 8):
                    acc = acc + w[k] * plsc.load_gather(g_v, [rowk[k], col])
                return acc

            sigma = jnp.maximum(interp(0), 0.0)
            sigma_v[pl.ds(base, _LANES)] = sigma

            for c in range(3):
                acc = basis[0] * interp(1 + c * 9)
                for b in range(1, 9):
                    acc = acc + basis[b] * interp(1 + c * 9 + b)
                color_v[c, pl.ds(base, _LANES)] = acc
            return c2

        lax.fori_loop(0, _NGRP, compute_group, 0)

        for c in range(3):
            pltpu.sync_copy(color_v.at[c],
                            color_hbm.at[pl.ds(c * _P + pbase, _BC)])
        pltpu.sync_copy(sigma_v, sigma_hbm.at[pl.ds(pbase, _BC)])
        return carry

    lax.fori_loop(0, _NCHUNK, chunk_body, 0)


@jax.jit
def kernel(x, d, voxel_grid):
    xt = x.T.reshape(-1)
    dt = d.T.reshape(-1)
    tab = jnp.pad(voxel_grid.reshape(_G * _G * _G, _C),
                  ((0, 0), (0, _CT - _C)))
    mesh = plsc.VectorSubcoreMesh(core_axis_name="c", subcore_axis_name="s",
                                  num_cores=_NC, num_subcores=_NS)
    fn = pl.kernel(
        _sc_body,
        out_type=(
            jax.ShapeDtypeStruct((3 * _P,), jnp.float32),
            jax.ShapeDtypeStruct((_P,), jnp.float32),
        ),
        mesh=mesh,
        compiler_params=pltpu.CompilerParams(needs_layout_passes=False,
                                             use_tc_tiling_on_sc=False),
        scratch_types=[
            pltpu.VMEM((3, _BC), jnp.float32),
            pltpu.VMEM((3, _BC), jnp.float32),
            pltpu.VMEM((8, _BC), jnp.int32),
            pltpu.VMEM((8, _BC), jnp.float32),
            pltpu.VMEM((8 * _BC, _CT), jnp.float32),
            pltpu.VMEM((3, _BC), jnp.float32),
            pltpu.VMEM((_BC,), jnp.float32),
            pltpu.SemaphoreType.DMA,
        ],
    )
    color_t, sigma = fn(xt, dt, tab)
    color = color_t.reshape(3, _P).T
    return (color, sigma)


# speedup vs baseline: 3.3861x; 1.0706x over previous
import jax
import jax.numpy as jnp
from jax import lax
from jax.experimental import pallas as pl
from jax.experimental.pallas import tpu as pltpu
from jax.experimental.pallas import tpu_sc as plsc

_SH_C0 = 0.28209479177387814
_SH_C1 = 0.4886025119029199
_SH_C2 = (1.0925484305920792, -1.0925484305920792, 0.31539156525252005,
          -1.0925484305920792, 0.5462742152960396)

_SCALE = 1.5
_G = 128
_P = 524288
_C = 28
_CT = 32

_NC = 2
_NS = 16
_NW = _NC * _NS
_BC = 128
_PPW = _P // _NW
_NCHUNK = _PPW // _BC

_LANES = 16
_NGRP = _BC // _LANES


def _axis_setup(coord):
    g = coord / _SCALE
    ix = ((g + 1.0) * float(_G) - 1.0) * 0.5
    ixc = jnp.clip(ix, -1.0, float(_G))
    t = ixc.astype(jnp.int32)
    tf = t.astype(jnp.float32)
    fl = tf - jnp.where(tf > ixc, 1.0, 0.0)
    fli = fl.astype(jnp.int32)
    frac = ixc - fl
    v0 = (ix >= 0.0) & (ix < float(_G))
    v1 = (ix >= -1.0) & (ix < float(_G) - 1.0)
    c0 = jnp.clip(fli, 0, _G - 1)
    c1 = jnp.clip(fli + 1, 0, _G - 1)
    w0 = jnp.where(v0, 1.0 - frac, 0.0)
    w1 = jnp.where(v1, frac, 0.0)
    return c0, c1, w0, w1


def _sc_body(x_hbm, d_hbm, tab_hbm, color_hbm, sigma_hbm,
             x_v, d_v, idx_v, w_v, g_v, color_v, sigma_v,
             gsem0, gsem1, osem0, osem1):
    wid = lax.axis_index("s") * _NC + lax.axis_index("c")
    lanes = lax.iota(jnp.int32, _LANES)
    zeros16 = jnp.zeros((_LANES,), jnp.int32)
    gsem = (gsem0, gsem1)
    osem = (osem0, osem1)

    def prep(ci, b):
        pbase = wid * _PPW + ci * _BC
        for c in range(3):
            pltpu.sync_copy(x_hbm.at[pl.ds(c * _P + pbase, _BC)], x_v.at[c])
            pltpu.sync_copy(d_hbm.at[pl.ds(c * _P + pbase, _BC)],
                            d_v.at[b, c])

        def prep_group(g, c2):
            base = g * _LANES
            px = x_v[0, pl.ds(base, _LANES)]
            py = x_v[1, pl.ds(base, _LANES)]
            pz = x_v[2, pl.ds(base, _LANES)]
            rows = base + lanes

            keep = ((px < _SCALE) & (px > -_SCALE) &
                    (py < _SCALE) & (py > -_SCALE) &
                    (pz < _SCALE) & (pz > -_SCALE))
            keep = keep | ((pbase + rows) == 0)
            keepf = jnp.where(keep, 1.0, 0.0)

            cx0, cx1, wx0, wx1 = _axis_setup(px)
            cy0, cy1, wy0, wy1 = _axis_setup(py)
            cz0, cz1, wz0, wz1 = _axis_setup(pz)
            wz0 = wz0 * keepf
            wz1 = wz1 * keepf

            z0 = cz0 * (_G * _G)
            z1 = cz1 * (_G * _G)
            y0 = cy0 * _G
            y1 = cy1 * _G
            wzy = (wz0 * wy0, wz0 * wy1, wz1 * wy0, wz1 * wy1)
            zy = (z0 + y0, z0 + y1, z1 + y0, z1 + y1)
            k = 0
            for j in range(4):
                for (cx, wx) in ((cx0, wx0), (cx1, wx1)):
                    idx_v[b, k, pl.ds(base, _LANES)] = zy[j] + cx
                    w_v[b, k, pl.ds(base, _LANES)] = wzy[j] * wx
                    k += 1
            return c2

        lax.fori_loop(0, _NGRP, prep_group, 0)
        for k in range(8):
            pltpu.async_copy(tab_hbm.at[idx_v.at[b, k]],
                             g_v.at[b, pl.ds(k * _BC, _BC)], gsem[b])

    def compute(ci, b):
        pbase = wid * _PPW + ci * _BC
        for k in range(8):
            pltpu.make_async_copy(tab_hbm.at[idx_v.at[b, k]],
                                  g_v.at[b, pl.ds(k * _BC, _BC)],
                                  gsem[b]).wait()

        @pl.when(ci >= 2)
        def _():
            for c in range(3):
                pltpu.make_async_copy(
                    color_v.at[b, c],
                    color_hbm.at[pl.ds(c * _P + pbase, _BC)],
                    osem[b]).wait()
            pltpu.make_async_copy(
                sigma_v.at[b], sigma_hbm.at[pl.ds(pbase, _BC)],
                osem[b]).wait()

        def compute_group(g, c2):
            base = g * _LANES
            rows = base + lanes
            dx = d_v[b, 0, pl.ds(base, _LANES)]
            dy = d_v[b, 1, pl.ds(base, _LANES)]
            dz = d_v[b, 2, pl.ds(base, _LANES)]
            xx, yy, zz = dx * dx, dy * dy, dz * dz
            basis = (
                jnp.full((_LANES,), _SH_C0, jnp.float32),
                (-_SH_C1) * dy,
                _SH_C1 * dz,
                (-_SH_C1) * dx,
                _SH_C2[0] * (dx * dy),
                _SH_C2[1] * (dy * dz),
                _SH_C2[2] * (2.0 * zz - xx - yy),
                _SH_C2[3] * (dx * dz),
                _SH_C2[4] * (xx - yy),
            )
            w = [w_v[b, k, pl.ds(base, _LANES)] for k in range(8)]
            rowk = [k * _BC + rows for k in range(8)]
            gb = g_v.at[b]

            def interp(ch):
                col = zeros16 + ch
                acc = w[0] * plsc.load_gather(gb, [rowk[0], col])
                for k in range(1, 8):
                    acc = acc + w[k] * plsc.load_gather(gb, [rowk[k], col])
                return acc

            sigma = jnp.maximum(interp(0), 0.0)
            sigma_v[b, pl.ds(base, _LANES)] = sigma

            for c in range(3):
                acc = basis[0] * interp(1 + c * 9)
                for bb in range(1, 9):
                    acc = acc + basis[bb] * interp(1 + c * 9 + bb)
                color_v[b, c, pl.ds(base, _LANES)] = acc
            return c2

        lax.fori_loop(0, _NGRP, compute_group, 0)

        for c in range(3):
            pltpu.async_copy(color_v.at[b, c],
                             color_hbm.at[pl.ds(c * _P + pbase, _BC)],
                             osem[b])
        pltpu.async_copy(sigma_v.at[b], sigma_hbm.at[pl.ds(pbase, _BC)],
                         osem[b])

    prep(0, 0)

    def pair_body(cc, carry):
        for b in (0, 1):
            ci = 2 * cc + b

            @pl.when(ci + 1 < _NCHUNK)
            def _():
                prep(ci + 1, 1 - b)

            compute(ci, b)
        return carry

    lax.fori_loop(0, _NCHUNK // 2, pair_body, 0)

    for b in (0, 1):
        for c in range(3):
            pltpu.make_async_copy(color_v.at[b, c],
                                  color_hbm.at[pl.ds(c * _P, _BC)],
                                  osem[b]).wait()
        pltpu.make_async_copy(sigma_v.at[b], sigma_hbm.at[pl.ds(0, _BC)],
                              osem[b]).wait()


@jax.jit
def kernel(x, d, voxel_grid):
    xt = x.T.reshape(-1)
    dt = d.T.reshape(-1)
    tab = jnp.pad(voxel_grid.reshape(_G * _G * _G, _C),
                  ((0, 0), (0, _CT - _C)))
    mesh = plsc.VectorSubcoreMesh(core_axis_name="c", subcore_axis_name="s",
                                  num_cores=_NC, num_subcores=_NS)
    fn = pl.kernel(
        _sc_body,
        out_type=(
            jax.ShapeDtypeStruct((3 * _P,), jnp.float32),
            jax.ShapeDtypeStruct((_P,), jnp.float32),
        ),
        mesh=mesh,
        compiler_params=pltpu.CompilerParams(needs_layout_passes=False,
                                             use_tc_tiling_on_sc=False),
        scratch_types=[
            pltpu.VMEM((3, _BC), jnp.float32),
            pltpu.VMEM((2, 3, _BC), jnp.float32),
            pltpu.VMEM((2, 8, _BC), jnp.int32),
            pltpu.VMEM((2, 8, _BC), jnp.float32),
            pltpu.VMEM((2, 8 * _BC, _CT), jnp.float32),
            pltpu.VMEM((2, 3, _BC), jnp.float32),
            pltpu.VMEM((2, _BC), jnp.float32),
            pltpu.SemaphoreType.DMA,
            pltpu.SemaphoreType.DMA,
            pltpu.SemaphoreType.DMA,
            pltpu.SemaphoreType.DMA,
        ],
    )
    color_t, sigma = fn(xt, dt, tab)
    color = color_t.reshape(3, _P).T
    return (color, sigma)


# speedup vs baseline: 3.7922x; 1.1199x over previous
import jax
import jax.numpy as jnp
from jax import lax
from jax.experimental import pallas as pl
from jax.experimental.pallas import tpu as pltpu
from jax.experimental.pallas import tpu_sc as plsc

_SH_C0 = 0.28209479177387814
_SH_C1 = 0.4886025119029199
_SH_C2 = (1.0925484305920792, -1.0925484305920792, 0.31539156525252005,
          -1.0925484305920792, 0.5462742152960396)

_SCALE = 1.5
_G = 128
_P = 524288
_C = 28
_CT = 32

_NC = 2
_NS = 16
_NW = _NC * _NS
_BC = 128
_PPW = _P // _NW
_NCHUNK = _PPW // _BC

_LANES = 16
_NGRP = _BC // _LANES


def _axis_setup(coord):
    g = coord / _SCALE
    ix = ((g + 1.0) * float(_G) - 1.0) * 0.5
    ixc = jnp.clip(ix, -1.0, float(_G))
    t = ixc.astype(jnp.int32)
    tf = t.astype(jnp.float32)
    fl = tf - jnp.where(tf > ixc, 1.0, 0.0)
    fli = fl.astype(jnp.int32)
    frac = ixc - fl
    v0 = (ix >= 0.0) & (ix < float(_G))
    v1 = (ix >= -1.0) & (ix < float(_G) - 1.0)
    c0 = jnp.clip(fli, 0, _G - 1)
    c1 = jnp.clip(fli + 1, 0, _G - 1)
    w0 = jnp.where(v0, 1.0 - frac, 0.0)
    w1 = jnp.where(v1, frac, 0.0)
    return c0, c1, w0, w1


_RB = 8
_RBLK = _G // _RB
_APW = _G // _NW


def _repack_body(vt_hbm, tab_hbm, in_v, out_v, isem0, isem1, osem0, osem1):
    wid = lax.axis_index("s") * _NC + lax.axis_index("c")
    lanes = lax.iota(jnp.int32, _LANES)
    isem = (isem0, isem1)
    osem = (osem0, osem1)

    nblk = _APW * _RBLK

    def fire(t, buf):
        a = wid * _APW + t // _RBLK
        blk = t % _RBLK
        for ch in range(_C):
            pltpu.async_copy(vt_hbm.at[a, ch, pl.ds(blk * _RB, _RB)],
                             in_v.at[buf, ch], isem[buf])

    def flush(t, buf):
        a = wid * _APW + t // _RBLK
        blk = t % _RBLK
        for ch in range(_C):
            pltpu.make_async_copy(vt_hbm.at[a, ch, pl.ds(blk * _RB, _RB)],
                                  in_v.at[buf, ch], isem[buf]).wait()

        def per_ch(ch, c2):
            col = jnp.zeros((_LANES,), jnp.int32) + ch
            for bb in range(_RB):
                for c0 in range(0, _G, _LANES):
                    val = in_v[buf, ch, bb, pl.ds(c0, _LANES)]
                    plsc.store_scatter(out_v.at[buf],
                                       [bb * _G + c0 + lanes, col], val)
            return c2

        lax.fori_loop(0, _C, per_ch, 0)
        rowbase = a * (_G * _G) + blk * (_RB * _G)
        pltpu.async_copy(out_v.at[buf],
                         tab_hbm.at[pl.ds(rowbase, _RB * _G)], osem[buf])

    def drain_out(buf):
        pltpu.make_async_copy(out_v.at[buf],
                              tab_hbm.at[pl.ds(0, _RB * _G)],
                              osem[buf]).wait()

    fire(0, 0)

    def pair_body(cc, carry):
        for buf in (0, 1):
            t = 2 * cc + buf

            @pl.when(t + 1 < nblk)
            def _():
                fire(t + 1, 1 - buf)

            @pl.when(t >= 2)
            def _():
                drain_out(buf)

            flush(t, buf)
        return carry

    lax.fori_loop(0, nblk // 2, pair_body, 0)
    for buf in (0, 1):
        drain_out(buf)


def _sc_body(x_hbm, d_hbm, tab_hbm, color_hbm, sigma_hbm,
             x_v, d_v, idx_v, w_v, g_v, color_v, sigma_v,
             gsem0, gsem1, osem0, osem1):
    wid = lax.axis_index("s") * _NC + lax.axis_index("c")
    lanes = lax.iota(jnp.int32, _LANES)
    zeros16 = jnp.zeros((_LANES,), jnp.int32)
    gsem = (gsem0, gsem1)
    osem = (osem0, osem1)

    def prep(ci, b):
        pbase = wid * _PPW + ci * _BC
        for c in range(3):
            pltpu.sync_copy(x_hbm.at[pl.ds(c * _P + pbase, _BC)], x_v.at[c])
            pltpu.sync_copy(d_hbm.at[pl.ds(c * _P + pbase, _BC)],
                            d_v.at[b, c])

        def prep_group(g, c2):
            base = g * _LANES
            px = x_v[0, pl.ds(base, _LANES)]
            py = x_v[1, pl.ds(base, _LANES)]
            pz = x_v[2, pl.ds(base, _LANES)]
            rows = base + lanes

            keep = ((px < _SCALE) & (px > -_SCALE) &
                    (py < _SCALE) & (py > -_SCALE) &
                    (pz < _SCALE) & (pz > -_SCALE))
            keep = keep | ((pbase + rows) == 0)
            keepf = jnp.where(keep, 1.0, 0.0)

            cx0, cx1, wx0, wx1 = _axis_setup(px)
            cy0, cy1, wy0, wy1 = _axis_setup(py)
            cz0, cz1, wz0, wz1 = _axis_setup(pz)
            wz0 = wz0 * keepf
            wz1 = wz1 * keepf

            z0 = cz0 * (_G * _G)
            z1 = cz1 * (_G * _G)
            y0 = cy0 * _G
            y1 = cy1 * _G
            wzy = (wz0 * wy0, wz0 * wy1, wz1 * wy0, wz1 * wy1)
            zy = (z0 + y0, z0 + y1, z1 + y0, z1 + y1)
            k = 0
            for j in range(4):
                for (cx, wx) in ((cx0, wx0), (cx1, wx1)):
                    idx_v[b, k, pl.ds(base, _LANES)] = zy[j] + cx
                    w_v[b, k, pl.ds(base, _LANES)] = wzy[j] * wx
                    k += 1
            return c2

        lax.fori_loop(0, _NGRP, prep_group, 0)
        for k in range(8):
            pltpu.async_copy(tab_hbm.at[idx_v.at[b, k]],
                             g_v.at[b, pl.ds(k * _BC, _BC)], gsem[b])

    def compute(ci, b):
        pbase = wid * _PPW + ci * _BC
        for k in range(8):
            pltpu.make_async_copy(tab_hbm.at[idx_v.at[b, k]],
                                  g_v.at[b, pl.ds(k * _BC, _BC)],
                                  gsem[b]).wait()

        @pl.when(ci >= 2)
        def _():
            for c in range(3):
                pltpu.make_async_copy(
                    color_v.at[b, c],
                    color_hbm.at[pl.ds(c * _P + pbase, _BC)],
                    osem[b]).wait()
            pltpu.make_async_copy(
                sigma_v.at[b], sigma_hbm.at[pl.ds(pbase, _BC)],
                osem[b]).wait()

        def compute_group(g, c2):
            base = g * _LANES
            rows = base + lanes
            dx = d_v[b, 0, pl.ds(base, _LANES)]
            dy = d_v[b, 1, pl.ds(base, _LANES)]
            dz = d_v[b, 2, pl.ds(base, _LANES)]
            xx, yy, zz = dx * dx, dy * dy, dz * dz
            basis = (
                jnp.full((_LANES,), _SH_C0, jnp.float32),
                (-_SH_C1) * dy,
                _SH_C1 * dz,
                (-_SH_C1) * dx,
                _SH_C2[0] * (dx * dy),
                _SH_C2[1] * (dy * dz),
                _SH_C2[2] * (2.0 * zz - xx - yy),
                _SH_C2[3] * (dx * dz),
                _SH_C2[4] * (xx - yy),
            )
            w = [w_v[b, k, pl.ds(base, _LANES)] for k in range(8)]
            rowk = [k * _BC + rows for k in range(8)]
            gb = g_v.at[b]

            def interp(ch):
                col = zeros16 + ch
                acc = w[0] * plsc.load_gather(gb, [rowk[0], col])
                for k in range(1, 8):
                    acc = acc + w[k] * plsc.load_gather(gb, [rowk[k], col])
                return acc

            sigma = jnp.maximum(interp(0), 0.0)
            sigma_v[b, pl.ds(base, _LANES)] = sigma

            for c in range(3):
                acc = basis[0] * interp(1 + c * 9)
                for bb in range(1, 9):
                    acc = acc + basis[bb] * interp(1 + c * 9 + bb)
                color_v[b, c, pl.ds(base, _LANES)] = acc
            return c2

        lax.fori_loop(0, _NGRP, compute_group, 0)

        for c in range(3):
            pltpu.async_copy(color_v.at[b, c],
                             color_hbm.at[pl.ds(c * _P + pbase, _BC)],
                             osem[b])
        pltpu.async_copy(sigma_v.at[b], sigma_hbm.at[pl.ds(pbase, _BC)],
                         osem[b])

    prep(0, 0)

    def pair_body(cc, carry):
        for b in (0, 1):
            ci = 2 * cc + b

            @pl.when(ci + 1 < _NCHUNK)
            def _():
                prep(ci + 1, 1 - b)

            compute(ci, b)
        return carry

    lax.fori_loop(0, _NCHUNK // 2, pair_body, 0)

    for b in (0, 1):
        for c in range(3):
            pltpu.make_async_copy(color_v.at[b, c],
                                  color_hbm.at[pl.ds(c * _P, _BC)],
                                  osem[b]).wait()
        pltpu.make_async_copy(sigma_v.at[b], sigma_hbm.at[pl.ds(0, _BC)],
                              osem[b]).wait()


@jax.jit
def kernel(x, d, voxel_grid):
    xt = x.T.reshape(-1)
    dt = d.T.reshape(-1)
    vt = jnp.transpose(voxel_grid, (0, 3, 1, 2))
    mesh = plsc.VectorSubcoreMesh(core_axis_name="c", subcore_axis_name="s",
                                  num_cores=_NC, num_subcores=_NS)
    repack = pl.kernel(
        _repack_body,
        out_type=jax.ShapeDtypeStruct((_G * _G * _G, _CT), jnp.float32),
        mesh=mesh,
        compiler_params=pltpu.CompilerParams(needs_layout_passes=False,
                                             use_tc_tiling_on_sc=False),
        scratch_types=[
            pltpu.VMEM((2, _C, _RB, _G), jnp.float32),
            pltpu.VMEM((2, _RB * _G, _CT), jnp.float32),
            pltpu.SemaphoreType.DMA,
            pltpu.SemaphoreType.DMA,
            pltpu.SemaphoreType.DMA,
            pltpu.SemaphoreType.DMA,
        ],
    )
    tab = repack(vt)
    fn = pl.kernel(
        _sc_body,
        out_type=(
            jax.ShapeDtypeStruct((3 * _P,), jnp.float32),
            jax.ShapeDtypeStruct((_P,), jnp.float32),
        ),
        mesh=mesh,
        compiler_params=pltpu.CompilerParams(needs_layout_passes=False,
                                             use_tc_tiling_on_sc=False),
        scratch_types=[
            pltpu.VMEM((3, _BC), jnp.float32),
            pltpu.VMEM((2, 3, _BC), jnp.float32),
            pltpu.VMEM((2, 8, _BC), jnp.int32),
            pltpu.VMEM((2, 8, _BC), jnp.float32),
            pltpu.VMEM((2, 8 * _BC, _CT), jnp.float32),
            pltpu.VMEM((2, 3, _BC), jnp.float32),
            pltpu.VMEM((2, _BC), jnp.float32),
            pltpu.SemaphoreType.DMA,
            pltpu.SemaphoreType.DMA,
            pltpu.SemaphoreType.DMA,
            pltpu.SemaphoreType.DMA,
        ],
    )
    color_t, sigma = fn(xt, dt, tab)
    color = color_t.reshape(3, _P).T
    return (color, sigma)


# speedup vs baseline: 5.3885x; 1.4209x over previous
import jax
import jax.numpy as jnp
from jax import lax
from jax.experimental import pallas as pl
from jax.experimental.pallas import tpu as pltpu
from jax.experimental.pallas import tpu_sc as plsc

_SH_C0 = 0.28209479177387814
_SH_C1 = 0.4886025119029199
_SH_C2 = (1.0925484305920792, -1.0925484305920792, 0.31539156525252005,
          -1.0925484305920792, 0.5462742152960396)

_SCALE = 1.5
_G = 128
_P = 524288
_C = 28
_CT = 32

_NC = 2
_NS = 16
_NW = _NC * _NS
_BC = 128
_PPW = _P // _NW
_NCHUNK = _PPW // _BC

_LANES = 16
_NGRP = _BC // _LANES


def _axis_setup(coord):
    g = coord / _SCALE
    ix = ((g + 1.0) * float(_G) - 1.0) * 0.5
    ixc = jnp.clip(ix, -1.0, float(_G))
    t = ixc.astype(jnp.int32)
    tf = t.astype(jnp.float32)
    fl = tf - jnp.where(tf > ixc, 1.0, 0.0)
    fli = fl.astype(jnp.int32)
    frac = ixc - fl
    v0 = (ix >= 0.0) & (ix < float(_G))
    v1 = (ix >= -1.0) & (ix < float(_G) - 1.0)
    c0 = jnp.clip(fli, 0, _G - 1)
    c1 = jnp.clip(fli + 1, 0, _G - 1)
    w0 = jnp.where(v0, 1.0 - frac, 0.0)
    w1 = jnp.where(v1, frac, 0.0)
    return c0, c1, w0, w1


_RB = 8
_RBLK = _G // _RB
_APW = _G // _NW


def _repack_body(vt_hbm, tab_hbm, in_v, out_v, isem0, isem1, osem0):
    wid = lax.axis_index("s") * _NC + lax.axis_index("c")
    lanes = lax.iota(jnp.int32, _LANES)
    isem = (isem0, isem1)

    nblk = _APW * _RBLK
    nrow = _RB * _G

    zero16 = jnp.zeros((_LANES,), jnp.float32)
    for buf in (0, 1):
        for ch in range(_C, _CT):
            def zrow(q, c2, buf=buf, ch=ch):
                in_v[buf, ch, pl.ds(q * _LANES, _LANES)] = zero16
                return c2
            lax.fori_loop(0, nrow // _LANES, zrow, 0)

    def fire(t, buf):
        a = wid * _APW + t // _RBLK
        blk = t % _RBLK
        for ch in range(_C):
            pltpu.async_copy(vt_hbm.at[a, ch, pl.ds(blk * nrow, nrow)],
                             in_v.at[buf, ch, pl.ds(0, nrow)], isem[buf])

    def flush(t, buf):
        a = wid * _APW + t // _RBLK
        blk = t % _RBLK
        for ch in range(_C):
            pltpu.make_async_copy(vt_hbm.at[a, ch, pl.ds(blk * nrow, nrow)],
                                  in_v.at[buf, ch, pl.ds(0, nrow)],
                                  isem[buf]).wait()

        in2 = in_v.at[buf]
        rows_lo = lanes
        rows_hi = lanes + _LANES

        def per_row16(q, c2):
            r0 = q * _LANES
            for rr in range(_LANES):
                pos = jnp.zeros((_LANES,), jnp.int32) + (r0 + rr)
                lo = plsc.load_gather(in2, [rows_lo, pos])
                hi = plsc.load_gather(in2, [rows_hi, pos])
                out_v[r0 + rr, pl.ds(0, _LANES)] = lo
                out_v[r0 + rr, pl.ds(_LANES, _LANES)] = hi
            return c2

        lax.fori_loop(0, nrow // _LANES, per_row16, 0)
        rowbase = a * (_G * _G) + blk * nrow
        pltpu.async_copy(out_v, tab_hbm.at[pl.ds(rowbase, nrow)], osem0)

    def drain_out():
        pltpu.make_async_copy(out_v, tab_hbm.at[pl.ds(0, nrow)],
                              osem0).wait()

    fire(0, 0)

    def pair_body(cc, carry):
        for buf in (0, 1):
            t = 2 * cc + buf

            @pl.when(t + 1 < nblk)
            def _():
                fire(t + 1, 1 - buf)

            @pl.when(t >= 1)
            def _():
                drain_out()

            flush(t, buf)
        return carry

    lax.fori_loop(0, nblk // 2, pair_body, 0)
    drain_out()


def _sc_body(x_hbm, d_hbm, tab_hbm, color_hbm, sigma_hbm,
             x_v, d_v, idx_v, w_v, g_v, color_v, sigma_v, s_v,
             gsem0, gsem1, osem0, osem1):
    wid = lax.axis_index("s") * _NC + lax.axis_index("c")
    lanes = lax.iota(jnp.int32, _LANES)
    zeros16 = jnp.zeros((_LANES,), jnp.int32)
    gsem = (gsem0, gsem1)
    osem = (osem0, osem1)

    def prep(ci, b):
        pbase = wid * _PPW + ci * _BC
        for c in range(3):
            pltpu.sync_copy(x_hbm.at[pl.ds(c * _P + pbase, _BC)], x_v.at[c])
            pltpu.sync_copy(d_hbm.at[pl.ds(c * _P + pbase, _BC)],
                            d_v.at[b, c])

        def prep_group(g, c2):
            base = g * _LANES
            px = x_v[0, pl.ds(base, _LANES)]
            py = x_v[1, pl.ds(base, _LANES)]
            pz = x_v[2, pl.ds(base, _LANES)]
            rows = base + lanes

            keep = ((px < _SCALE) & (px > -_SCALE) &
                    (py < _SCALE) & (py > -_SCALE) &
                    (pz < _SCALE) & (pz > -_SCALE))
            keep = keep | ((pbase + rows) == 0)
            keepf = jnp.where(keep, 1.0, 0.0)

            cx0, cx1, wx0, wx1 = _axis_setup(px)
            cy0, cy1, wy0, wy1 = _axis_setup(py)
            cz0, cz1, wz0, wz1 = _axis_setup(pz)
            wz0 = wz0 * keepf
            wz1 = wz1 * keepf

            z0 = cz0 * (_G * _G)
            z1 = cz1 * (_G * _G)
            y0 = cy0 * _G
            y1 = cy1 * _G
            wzy = (wz0 * wy0, wz0 * wy1, wz1 * wy0, wz1 * wy1)
            zy = (z0 + y0, z0 + y1, z1 + y0, z1 + y1)
            k = 0
            for j in range(4):
                for (cx, wx) in ((cx0, wx0), (cx1, wx1)):
                    idx_v[b, k, pl.ds(base, _LANES)] = zy[j] + cx
                    w_v[b, k, pl.ds(base, _LANES)] = wzy[j] * wx
                    k += 1
            return c2

        lax.fori_loop(0, _NGRP, prep_group, 0)
        for k in range(8):
            pltpu.async_copy(tab_hbm.at[idx_v.at[b, k]],
                             g_v.at[b, pl.ds(k * _BC, _BC)], gsem[b])

    def compute(ci, b):
        pbase = wid * _PPW + ci * _BC
        for k in range(8):
            pltpu.make_async_copy(tab_hbm.at[idx_v.at[b, k]],
                                  g_v.at[b, pl.ds(k * _BC, _BC)],
                                  gsem[b]).wait()

        @pl.when(ci >= 2)
        def _():
            for c in range(3):
                pltpu.make_async_copy(
                    color_v.at[b, c],
                    color_hbm.at[pl.ds(c * _P + pbase, _BC)],
                    osem[b]).wait()
            pltpu.make_async_copy(
                sigma_v.at[b], sigma_hbm.at[pl.ds(pbase, _BC)],
                osem[b]).wait()

        def compute_group(g, c2):
            base = g * _LANES
            rows = base + lanes
            dx = d_v[b, 0, pl.ds(base, _LANES)]
            dy = d_v[b, 1, pl.ds(base, _LANES)]
            dz = d_v[b, 2, pl.ds(base, _LANES)]
            xx, yy, zz = dx * dx, dy * dy, dz * dz
            basis = (
                jnp.full((_LANES,), _SH_C0, jnp.float32),
                (-_SH_C1) * dy,
                _SH_C1 * dz,
                (-_SH_C1) * dx,
                _SH_C2[0] * (dx * dy),
                _SH_C2[1] * (dy * dz),
                _SH_C2[2] * (2.0 * zz - xx - yy),
                _SH_C2[3] * (dx * dz),
                _SH_C2[4] * (xx - yy),
            )
            w = [w_v[b, k, pl.ds(base, _LANES)] for k in range(8)]
            rowk = [k * _BC + rows for k in range(8)]
            gb = g_v.at[b]

            col = lanes
            for j in range(_CT):
                acc = w[0] * plsc.load_gather(gb, [rowk[0], col])
                for k in range(1, 8):
                    acc = acc + w[k] * plsc.load_gather(gb, [rowk[k], col])
                s_v[j, pl.ds(0, _LANES)] = acc
                col = (col + 1) & (_CT - 1)

            def chan(ch):
                rws = (zeros16 + (_CT + ch) - lanes) & (_CT - 1)
                return plsc.load_gather(s_v, [rws, lanes])

            sigma = jnp.maximum(chan(0), 0.0)
            sigma_v[b, pl.ds(base, _LANES)] = sigma

            for c in range(3):
                acc = basis[0] * chan(1 + c * 9)
                for bb in range(1, 9):
                    acc = acc + basis[bb] * chan(1 + c * 9 + bb)
                color_v[b, c, pl.ds(base, _LANES)] = acc
            return c2

        lax.fori_loop(0, _NGRP, compute_group, 0)

        for c in range(3):
            pltpu.async_copy(color_v.at[b, c],
                             color_hbm.at[pl.ds(c * _P + pbase, _BC)],
                             osem[b])
        pltpu.async_copy(sigma_v.at[b], sigma_hbm.at[pl.ds(pbase, _BC)],
                         osem[b])

    prep(0, 0)

    def pair_body(cc, carry):
        for b in (0, 1):
            ci = 2 * cc + b

            @pl.when(ci + 1 < _NCHUNK)
            def _():
                prep(ci + 1, 1 - b)

            compute(ci, b)
        return carry

    lax.fori_loop(0, _NCHUNK // 2, pair_body, 0)

    for b in (0, 1):
        for c in range(3):
            pltpu.make_async_copy(color_v.at[b, c],
                                  color_hbm.at[pl.ds(c * _P, _BC)],
                                  osem[b]).wait()
        pltpu.make_async_copy(sigma_v.at[b], sigma_hbm.at[pl.ds(0, _BC)],
                              osem[b]).wait()


@jax.jit
def kernel(x, d, voxel_grid):
    xt = x.T.reshape(-1)
    dt = d.T.reshape(-1)
    vt = jnp.transpose(voxel_grid, (0, 3, 1, 2)).reshape(_G, _C, _G * _G)
    mesh = plsc.VectorSubcoreMesh(core_axis_name="c", subcore_axis_name="s",
                                  num_cores=_NC, num_subcores=_NS)
    repack = pl.kernel(
        _repack_body,
        out_type=jax.ShapeDtypeStruct((_G * _G * _G, _CT), jnp.float32),
        mesh=mesh,
        compiler_params=pltpu.CompilerParams(needs_layout_passes=False,
                                             use_tc_tiling_on_sc=False),
        scratch_types=[
            pltpu.VMEM((2, _CT, _RB * _G + 9), jnp.float32),
            pltpu.VMEM((_RB * _G, _CT), jnp.float32),
            pltpu.SemaphoreType.DMA,
            pltpu.SemaphoreType.DMA,
            pltpu.SemaphoreType.DMA,
        ],
    )
    tab = repack(vt)
    fn = pl.kernel(
        _sc_body,
        out_type=(
            jax.ShapeDtypeStruct((3 * _P,), jnp.float32),
            jax.ShapeDtypeStruct((_P,), jnp.float32),
        ),
        mesh=mesh,
        compiler_params=pltpu.CompilerParams(needs_layout_passes=False,
                                             use_tc_tiling_on_sc=False),
        scratch_types=[
            pltpu.VMEM((3, _BC), jnp.float32),
            pltpu.VMEM((2, 3, _BC), jnp.float32),
            pltpu.VMEM((2, 8, _BC), jnp.int32),
            pltpu.VMEM((2, 8, _BC), jnp.float32),
            pltpu.VMEM((2, 8 * _BC, _CT), jnp.float32),
            pltpu.VMEM((2, 3, _BC), jnp.float32),
            pltpu.VMEM((2, _BC), jnp.float32),
            pltpu.VMEM((_CT, _LANES), jnp.float32),
            pltpu.SemaphoreType.DMA,
            pltpu.SemaphoreType.DMA,
            pltpu.SemaphoreType.DMA,
            pltpu.SemaphoreType.DMA,
        ],
    )
    color_t, sigma = fn(xt, dt, tab)
    color = color_t.reshape(3, _P).T
    return (color, sigma)


# speedup vs baseline: 5.5655x; 1.0328x over previous
import jax
import jax.numpy as jnp
from jax import lax
from jax.experimental import pallas as pl
from jax.experimental.pallas import tpu as pltpu
from jax.experimental.pallas import tpu_sc as plsc

_SH_C0 = 0.28209479177387814
_SH_C1 = 0.4886025119029199
_SH_C2 = (1.0925484305920792, -1.0925484305920792, 0.31539156525252005,
          -1.0925484305920792, 0.5462742152960396)

_SCALE = 1.5
_G = 128
_P = 524288
_C = 28
_CT = 32

_NC = 2
_NS = 16
_NW = _NC * _NS
_BC = 128
_PPW = _P // _NW
_NCHUNK = _PPW // _BC

_LANES = 16
_NGRP = _BC // _LANES


def _axis_setup(coord):
    g = coord / _SCALE
    ix = ((g + 1.0) * float(_G) - 1.0) * 0.5
    ixc = jnp.clip(ix, -1.0, float(_G))
    t = ixc.astype(jnp.int32)
    tf = t.astype(jnp.float32)
    fl = tf - jnp.where(tf > ixc, 1.0, 0.0)
    fli = fl.astype(jnp.int32)
    frac = ixc - fl
    v0 = (ix >= 0.0) & (ix < float(_G))
    v1 = (ix >= -1.0) & (ix < float(_G) - 1.0)
    c0 = jnp.clip(fli, 0, _G - 1)
    c1 = jnp.clip(fli + 1, 0, _G - 1)
    w0 = jnp.where(v0, 1.0 - frac, 0.0)
    w1 = jnp.where(v1, frac, 0.0)
    return c0, c1, w0, w1


_RB = 8
_RBLK = _G // _RB
_APW = _G // _NW


def _repack_body(vt_hbm, tab_hbm, in_v, out_v, isem0, isem1, osem0, osem1):
    wid = lax.axis_index("s") * _NC + lax.axis_index("c")
    lanes = lax.iota(jnp.int32, _LANES)
    isem = (isem0, isem1)
    osem = (osem0, osem1)

    nblk = _APW * _RBLK
    nrow = _RB * _G

    zero16 = jnp.zeros((_LANES,), jnp.float32)
    for buf in (0, 1):
        for ch in range(_C, _CT):
            def zrow(q, c2, buf=buf, ch=ch):
                in_v[buf, ch, pl.ds(q * _LANES, _LANES)] = zero16
                return c2
            lax.fori_loop(0, nrow // _LANES, zrow, 0)

    def fire(t, buf):
        a = wid * _APW + t // _RBLK
        blk = t % _RBLK
        for ch in range(_C):
            pltpu.async_copy(vt_hbm.at[a, ch, pl.ds(blk * nrow, nrow)],
                             in_v.at[buf, ch, pl.ds(0, nrow)], isem[buf])

    def flush(t, buf):
        a = wid * _APW + t // _RBLK
        blk = t % _RBLK
        for ch in range(_C):
            pltpu.make_async_copy(vt_hbm.at[a, ch, pl.ds(blk * nrow, nrow)],
                                  in_v.at[buf, ch, pl.ds(0, nrow)],
                                  isem[buf]).wait()

        in2 = in_v.at[buf]
        rows_lo = lanes
        rows_hi = lanes + _LANES
        half_rows = nrow // 2
        rowbase = a * (_G * _G) + blk * nrow

        for half in (0, 1):
            @pl.when(t >= 1)
            def _(half=half):
                drain_out(half)

            def per_row16(q, c2, half=half):
                r0 = q * _LANES
                for rr in range(_LANES):
                    pos = (jnp.zeros((_LANES,), jnp.int32)
                           + (half * half_rows + r0 + rr))
                    lo = plsc.load_gather(in2, [rows_lo, pos])
                    hi = plsc.load_gather(in2, [rows_hi, pos])
                    out_v[half, r0 + rr, pl.ds(0, _LANES)] = lo
                    out_v[half, r0 + rr, pl.ds(_LANES, _LANES)] = hi
                return c2

            lax.fori_loop(0, half_rows // _LANES, per_row16, 0)
            pltpu.async_copy(
                out_v.at[half],
                tab_hbm.at[pl.ds(rowbase + half * half_rows, half_rows)],
                osem[half])

    def drain_out(half):
        pltpu.make_async_copy(out_v.at[half],
                              tab_hbm.at[pl.ds(0, nrow // 2)],
                              osem[half]).wait()

    fire(0, 0)

    def pair_body(cc, carry):
        for buf in (0, 1):
            t = 2 * cc + buf

            @pl.when(t + 1 < nblk)
            def _():
                fire(t + 1, 1 - buf)

            flush(t, buf)
        return carry

    lax.fori_loop(0, nblk // 2, pair_body, 0)
    for half in (0, 1):
        drain_out(half)


def _sc_body(x_hbm, d_hbm, tab_hbm, color_hbm, sigma_hbm,
             x_v, d_v, idx_v, w_v, g_v, color_v, sigma_v, s_v,
             gsem0, gsem1, osem0, osem1):
    wid = lax.axis_index("s") * _NC + lax.axis_index("c")
    lanes = lax.iota(jnp.int32, _LANES)
    zeros16 = jnp.zeros((_LANES,), jnp.int32)
    gsem = (gsem0, gsem1)
    osem = (osem0, osem1)

    def prep(ci, b):
        pbase = wid * _PPW + ci * _BC
        for c in range(3):
            pltpu.sync_copy(x_hbm.at[pl.ds(c * _P + pbase, _BC)], x_v.at[c])
            pltpu.sync_copy(d_hbm.at[pl.ds(c * _P + pbase, _BC)],
                            d_v.at[b, c])

        def prep_group(g, c2):
            base = g * _LANES
            px = x_v[0, pl.ds(base, _LANES)]
            py = x_v[1, pl.ds(base, _LANES)]
            pz = x_v[2, pl.ds(base, _LANES)]
            rows = base + lanes

            keep = ((px < _SCALE) & (px > -_SCALE) &
                    (py < _SCALE) & (py > -_SCALE) &
                    (pz < _SCALE) & (pz > -_SCALE))
            keep = keep | ((pbase + rows) == 0)
            keepf = jnp.where(keep, 1.0, 0.0)

            cx0, cx1, wx0, wx1 = _axis_setup(px)
            cy0, cy1, wy0, wy1 = _axis_setup(py)
            cz0, cz1, wz0, wz1 = _axis_setup(pz)
            wz0 = wz0 * keepf
            wz1 = wz1 * keepf

            z0 = cz0 * (_G * _G)
            z1 = cz1 * (_G * _G)
            y0 = cy0 * _G
            y1 = cy1 * _G
            wzy = (wz0 * wy0, wz0 * wy1, wz1 * wy0, wz1 * wy1)
            zy = (z0 + y0, z0 + y1, z1 + y0, z1 + y1)
            k = 0
            for j in range(4):
                for (cx, wx) in ((cx0, wx0), (cx1, wx1)):
                    idx_v[b, k, pl.ds(base, _LANES)] = zy[j] + cx
                    w_v[b, k, pl.ds(base, _LANES)] = wzy[j] * wx
                    k += 1
            return c2

        lax.fori_loop(0, _NGRP, prep_group, 0)
        for k in range(8):
            pltpu.async_copy(tab_hbm.at[idx_v.at[b, k]],
                             g_v.at[b, pl.ds(k * _BC, _BC)], gsem[b])

    def compute(ci, b):
        pbase = wid * _PPW + ci * _BC
        for k in range(8):
            pltpu.make_async_copy(tab_hbm.at[idx_v.at[b, k]],
                                  g_v.at[b, pl.ds(k * _BC, _BC)],
                                  gsem[b]).wait()

        @pl.when(ci >= 2)
        def _():
            for c in range(3):
                pltpu.make_async_copy(
                    color_v.at[b, c],
                    color_hbm.at[pl.ds(c * _P + pbase, _BC)],
                    osem[b]).wait()
            pltpu.make_async_copy(
                sigma_v.at[b], sigma_hbm.at[pl.ds(pbase, _BC)],
                osem[b]).wait()

        def compute_group(g, c2):
            base = g * _LANES
            rows = base + lanes
            dx = d_v[b, 0, pl.ds(base, _LANES)]
            dy = d_v[b, 1, pl.ds(base, _LANES)]
            dz = d_v[b, 2, pl.ds(base, _LANES)]
            xx, yy, zz = dx * dx, dy * dy, dz * dz
            basis = (
                jnp.full((_LANES,), _SH_C0, jnp.float32),
                (-_SH_C1) * dy,
                _SH_C1 * dz,
                (-_SH_C1) * dx,
                _SH_C2[0] * (dx * dy),
                _SH_C2[1] * (dy * dz),
                _SH_C2[2] * (2.0 * zz - xx - yy),
                _SH_C2[3] * (dx * dz),
                _SH_C2[4] * (xx - yy),
            )
            w = [w_v[b, k, pl.ds(base, _LANES)] for k in range(8)]
            rowk = [k * _BC + rows for k in range(8)]
            gb = g_v.at[b]

            col = lanes
            for j in range(_CT):
                acc = w[0] * plsc.load_gather(gb, [rowk[0], col])
                for k in range(1, 8):
                    acc = acc + w[k] * plsc.load_gather(gb, [rowk[k], col])
                s_v[j, pl.ds(0, _LANES)] = acc
                col = (col + 1) & (_CT - 1)

            def chan(ch):
                rws = (zeros16 + (_CT + ch) - lanes) & (_CT - 1)
                return plsc.load_gather(s_v, [rws, lanes])

            sigma = jnp.maximum(chan(0), 0.0)
            sigma_v[b, pl.ds(base, _LANES)] = sigma

            for c in range(3):
                acc = basis[0] * chan(1 + c * 9)
                for bb in range(1, 9):
                    acc = acc + basis[bb] * chan(1 + c * 9 + bb)
                color_v[b, c, pl.ds(base, _LANES)] = acc
            return c2

        lax.fori_loop(0, _NGRP, compute_group, 0)

        for c in range(3):
            pltpu.async_copy(color_v.at[b, c],
                             color_hbm.at[pl.ds(c * _P + pbase, _BC)],
                             osem[b])
        pltpu.async_copy(sigma_v.at[b], sigma_hbm.at[pl.ds(pbase, _BC)],
                         osem[b])

    prep(0, 0)

    def pair_body(cc, carry):
        for b in (0, 1):
            ci = 2 * cc + b

            @pl.when(ci + 1 < _NCHUNK)
            def _():
                prep(ci + 1, 1 - b)

            compute(ci, b)
        return carry

    lax.fori_loop(0, _NCHUNK // 2, pair_body, 0)

    for b in (0, 1):
        for c in range(3):
            pltpu.make_async_copy(color_v.at[b, c],
                                  color_hbm.at[pl.ds(c * _P, _BC)],
                                  osem[b]).wait()
        pltpu.make_async_copy(sigma_v.at[b], sigma_hbm.at[pl.ds(0, _BC)],
                              osem[b]).wait()


@jax.jit
def kernel(x, d, voxel_grid):
    xt = x.T.reshape(-1)
    dt = d.T.reshape(-1)
    vt = jnp.transpose(voxel_grid, (0, 3, 1, 2)).reshape(_G, _C, _G * _G)
    mesh = plsc.VectorSubcoreMesh(core_axis_name="c", subcore_axis_name="s",
                                  num_cores=_NC, num_subcores=_NS)
    repack = pl.kernel(
        _repack_body,
        out_type=jax.ShapeDtypeStruct((_G * _G * _G, _CT), jnp.float32),
        mesh=mesh,
        compiler_params=pltpu.CompilerParams(needs_layout_passes=False,
                                             use_tc_tiling_on_sc=False),
        scratch_types=[
            pltpu.VMEM((2, _CT, _RB * _G + 9), jnp.float32),
            pltpu.VMEM((2, _RB * _G // 2, _CT), jnp.float32),
            pltpu.SemaphoreType.DMA,
            pltpu.SemaphoreType.DMA,
            pltpu.SemaphoreType.DMA,
            pltpu.SemaphoreType.DMA,
        ],
    )
    tab = repack(vt)
    fn = pl.kernel(
        _sc_body,
        out_type=(
            jax.ShapeDtypeStruct((3 * _P,), jnp.float32),
            jax.ShapeDtypeStruct((_P,), jnp.float32),
        ),
        mesh=mesh,
        compiler_params=pltpu.CompilerParams(needs_layout_passes=False,
                                             use_tc_tiling_on_sc=False),
        scratch_types=[
            pltpu.VMEM((3, _BC), jnp.float32),
            pltpu.VMEM((2, 3, _BC), jnp.float32),
            pltpu.VMEM((2, 8, _BC), jnp.int32),
            pltpu.VMEM((2, 8, _BC), jnp.float32),
            pltpu.VMEM((2, 8 * _BC, _CT), jnp.float32),
            pltpu.VMEM((2, 3, _BC), jnp.float32),
            pltpu.VMEM((2, _BC), jnp.float32),
            pltpu.VMEM((_CT, _LANES), jnp.float32),
            pltpu.SemaphoreType.DMA,
            pltpu.SemaphoreType.DMA,
            pltpu.SemaphoreType.DMA,
            pltpu.SemaphoreType.DMA,
        ],
    )
    color_t, sigma = fn(xt, dt, tab)
    color = color_t.reshape(3, _P).T
    return (color, sigma)
